# baseline (speedup 1.0000x reference)
import functools

import jax
import jax.numpy as jnp
from jax import lax
from jax.experimental import pallas as pl
from jax.experimental.pallas import tpu as pltpu
import jax.experimental.pallas.tpu_sc as plsc

DIM = 768
HEADS = 8
K = 8192
DHEAD = 96
DMEM = 96
B = 8
N = 576

KBLK = 1024
KSTEPS = K // KBLK
LANES = 128
SUBTILES = KBLK // LANES

NWORKERS = 32
ROWS_TOTAL = B * N * HEADS
ROWS_PER_W = ROWS_TOTAL // NWORKERS
IDX_CHUNK = 128
NCHUNKS = ROWS_PER_W // IDX_CHUNK


def _argmin_body(x_ref, cb_ref, idx_ref, val_scr, idx_scr):
    h = pl.program_id(1)
    k = pl.program_id(2)

    @pl.when(k == 0)
    def _init():
        val_scr[...] = jnp.full((N, LANES), jnp.inf, jnp.float32)
        idx_scr[...] = jnp.zeros((N, LANES), jnp.int32)

    x = x_ref[0, 0]
    cb = cb_ref[0]

    xe = lax.dot_general(x.astype(jnp.bfloat16), cb.astype(jnp.bfloat16),
                         (((1,), (1,)), ((), ())),
                         preferred_element_type=jnp.float32)
    ones = jnp.ones((1, DHEAD), jnp.float32)
    e2 = lax.dot_general(ones, cb * cb, (((1,), (1,)), ((), ())),
                         preferred_element_type=jnp.float32,
                         precision=lax.Precision.HIGHEST)
    scores = e2 - 2.0 * xe

    lane_iota = lax.broadcasted_iota(jnp.int32, (N, LANES), 1)
    for j in range(SUBTILES):
        sj = scores[:, j * LANES:(j + 1) * LANES]
        base = k * KBLK + j * LANES
        cand_idx = lane_iota + base
        better = sj < val_scr[...]
        val_scr[...] = jnp.where(better, sj, val_scr[...])
        idx_scr[...] = jnp.where(better, cand_idx, idx_scr[...])

    @pl.when(k == KSTEPS - 1)
    def _finalize():
        vals = val_scr[...]
        idxs = idx_scr[...]
        rowmin = jnp.min(vals, axis=1, keepdims=True)
        big = jnp.int32(1 << 30)
        best = jnp.min(jnp.where(vals == rowmin, idxs, big),
                       axis=1, keepdims=True)
        idx_ref[0, 0] = best + h * K


def _compute_indices(xh, codebook):
    return pl.pallas_call(
        _argmin_body,
        grid=(B, HEADS, KSTEPS),
        in_specs=[
            pl.BlockSpec((1, 1, N, DHEAD), lambda b, h, k: (b, h, 0, 0)),
            pl.BlockSpec((1, KBLK, DHEAD), lambda b, h, k: (h, k, 0)),
        ],
        out_specs=pl.BlockSpec((1, 1, N, 1), lambda b, h, k: (b, h, 0, 0)),
        out_shape=jax.ShapeDtypeStruct((B, HEADS, N, 1), jnp.int32),
        scratch_shapes=[
            pltpu.VMEM((N, LANES), jnp.float32),
            pltpu.VMEM((N, LANES), jnp.int32),
        ],
    )(xh, codebook)


def _gather_body(table_hbm, idx_hbm, out_hbm, idx_v, rows_v, sem):
    wid = lax.axis_index("s") * 2 + lax.axis_index("c")
    pltpu.sync_copy(idx_hbm.at[wid], idx_v)
    copies = []
    for j in range(NCHUNKS):
        copies.append(pltpu.async_copy(
            table_hbm.at[idx_v.at[j]],
            rows_v.at[pl.ds(j * IDX_CHUNK, IDX_CHUNK)],
            sem))
    for c in copies:
        c.wait()
    pltpu.sync_copy(rows_v, out_hbm.at[pl.ds(wid * ROWS_PER_W, ROWS_PER_W)])


@functools.cache
def _gather_rows_kernel():
    return pl.kernel(
        _gather_body,
        out_type=jax.ShapeDtypeStruct((ROWS_TOTAL, DMEM), jnp.float32),
        mesh=plsc.VectorSubcoreMesh(core_axis_name="c", subcore_axis_name="s",
                                    num_cores=2, num_subcores=16),
        scratch_types=[
            pltpu.VMEM((NCHUNKS, IDX_CHUNK), jnp.int32),
            pltpu.VMEM((ROWS_PER_W, DMEM), jnp.float32),
            pltpu.SemaphoreType.DMA,
        ],
        compiler_params=pltpu.CompilerParams(use_tc_tiling_on_sc=False),
    )


def kernel(x, codebook, values):
    xh = x.reshape(B, N, HEADS, DHEAD).transpose(0, 2, 1, 3)
    idx = _compute_indices(xh, codebook)
    idx_flat = idx.reshape(B, HEADS, N).transpose(0, 2, 1)
    idx3 = idx_flat.reshape(NWORKERS, NCHUNKS, IDX_CHUNK)
    table = values.reshape(HEADS * K, DMEM)
    rows = _gather_rows_kernel()(table, idx3)
    return rows.reshape(B, N, HEADS * DMEM)


# speedup vs baseline: 1.8905x; 1.8905x over previous
import functools

import jax
import jax.numpy as jnp
from jax import lax
from jax.experimental import pallas as pl
from jax.experimental.pallas import tpu as pltpu
import jax.experimental.pallas.tpu_sc as plsc

DIM = 768
HEADS = 8
K = 8192
DHEAD = 96
DMEM = 96
B = 8
N = 576

KBLK = 1024
KSTEPS = K // KBLK
LANES = 128
SUBTILES = KBLK // LANES

NWORKERS = 32
ROWS_TOTAL = B * N * HEADS
ROWS_PER_W = ROWS_TOTAL // NWORKERS
IDX_CHUNK = 128
NCHUNKS = ROWS_PER_W // IDX_CHUNK


def _e2_body(cb_ref, e2_ref):
    cb = cb_ref[0]
    ones = jnp.ones((1, DHEAD), jnp.float32)
    e2_ref[0] = lax.dot_general(ones, cb * cb, (((1,), (1,)), ((), ())),
                                preferred_element_type=jnp.float32,
                                precision=lax.Precision.HIGHEST)


def _compute_e2(codebook):
    return pl.pallas_call(
        _e2_body,
        grid=(HEADS,),
        in_specs=[pl.BlockSpec((1, K, DHEAD), lambda h: (h, 0, 0))],
        out_specs=pl.BlockSpec((1, 1, K), lambda h: (h, 0, 0)),
        out_shape=jax.ShapeDtypeStruct((HEADS, 1, K), jnp.float32),
    )(codebook)


def _argmin_body(x_ref, cb_ref, e2_ref, idx_ref, val_scr, idx_scr):
    k = pl.program_id(1)

    @pl.when(k == 0)
    def _init():
        val_scr[...] = jnp.full((HEADS, N, LANES), jnp.inf, jnp.float32)
        idx_scr[...] = jnp.zeros((HEADS, N, LANES), jnp.int32)

    lane_iota = lax.broadcasted_iota(jnp.int32, (N, LANES), 1)
    finals = []
    for h in range(HEADS):
        xh = x_ref[0, :, h * DHEAD:(h + 1) * DHEAD]
        cbh = cb_ref[h]
        xen = lax.dot_general(xh, cbh, (((1,), (1,)), ((), ())),
                              preferred_element_type=jnp.float32)
        scores = xen + e2_ref[h]
        run_val = val_scr[h]
        run_idx = idx_scr[h]
        for j in range(SUBTILES):
            sj = scores[:, j * LANES:(j + 1) * LANES]
            cand = lane_iota + (k * KBLK + j * LANES)
            better = sj < run_val
            run_val = jnp.where(better, sj, run_val)
            run_idx = jnp.where(better, cand, run_idx)
        val_scr[h] = run_val
        idx_scr[h] = run_idx

        @pl.when(k == KSTEPS - 1)
        def _final(h=h, run_val=run_val, run_idx=run_idx):
            rowmin = jnp.min(run_val, axis=1, keepdims=True)
            big = jnp.int32(1 << 30)
            best = jnp.min(jnp.where(run_val == rowmin, run_idx, big),
                           axis=1, keepdims=True)
            idx_ref[0, h] = best + h * K


def _compute_indices(xneg2, cb_bf, e2):
    return pl.pallas_call(
        _argmin_body,
        grid=(B, KSTEPS),
        in_specs=[
            pl.BlockSpec((1, N, DIM), lambda b, k: (b, 0, 0)),
            pl.BlockSpec((HEADS, KBLK, DHEAD), lambda b, k: (0, k, 0)),
            pl.BlockSpec((HEADS, 1, KBLK), lambda b, k: (0, 0, k)),
        ],
        out_specs=pl.BlockSpec((1, HEADS, N, 1), lambda b, k: (b, 0, 0, 0)),
        out_shape=jax.ShapeDtypeStruct((B, HEADS, N, 1), jnp.int32),
        scratch_shapes=[
            pltpu.VMEM((HEADS, N, LANES), jnp.float32),
            pltpu.VMEM((HEADS, N, LANES), jnp.int32),
        ],
    )(xneg2, cb_bf, e2)


def _gather_body(table_hbm, idx_hbm, out_hbm, idx_v, rows_v, sem):
    wid = lax.axis_index("s") * 2 + lax.axis_index("c")
    pltpu.sync_copy(idx_hbm.at[wid], idx_v)
    copies = []
    for j in range(NCHUNKS):
        copies.append(pltpu.async_copy(
            table_hbm.at[idx_v.at[j]],
            rows_v.at[pl.ds(j * IDX_CHUNK, IDX_CHUNK)],
            sem))
    for c in copies:
        c.wait()
    pltpu.sync_copy(rows_v, out_hbm.at[pl.ds(wid * ROWS_PER_W, ROWS_PER_W)])


@functools.cache
def _gather_rows_kernel():
    return pl.kernel(
        _gather_body,
        out_type=jax.ShapeDtypeStruct((ROWS_TOTAL, DMEM), jnp.float32),
        mesh=plsc.VectorSubcoreMesh(core_axis_name="c", subcore_axis_name="s",
                                    num_cores=2, num_subcores=16),
        scratch_types=[
            pltpu.VMEM((NCHUNKS, IDX_CHUNK), jnp.int32),
            pltpu.VMEM((ROWS_PER_W, DMEM), jnp.float32),
            pltpu.SemaphoreType.DMA,
        ],
        compiler_params=pltpu.CompilerParams(use_tc_tiling_on_sc=False),
    )


def kernel(x, codebook, values):
    xneg2 = (-2.0 * x).astype(jnp.bfloat16)
    cb_bf = codebook.astype(jnp.bfloat16)
    e2 = _compute_e2(codebook)
    idx = _compute_indices(xneg2, cb_bf, e2)
    idx_flat = idx.reshape(B, HEADS, N).transpose(0, 2, 1)
    idx3 = idx_flat.reshape(NWORKERS, NCHUNKS, IDX_CHUNK)
    table = values.reshape(HEADS * K, DMEM)
    rows = _gather_rows_kernel()(table, idx3)
    return rows.reshape(B, N, HEADS * DMEM)


# speedup vs baseline: 2.1585x; 1.1418x over previous
import functools

import jax
import jax.numpy as jnp
from jax import lax
from jax.experimental import pallas as pl
from jax.experimental.pallas import tpu as pltpu
import jax.experimental.pallas.tpu_sc as plsc

DIM = 768
HEADS = 8
K = 8192
DHEAD = 96
DMEM = 96
B = 8
N = 576

KBLK = 2048
KSTEPS = K // KBLK
LANES = 128
SUBTILES = KBLK // LANES

NWORKERS = 32
ROWS_TOTAL = B * N * HEADS
ROWS_PER_W = ROWS_TOTAL // NWORKERS
IDX_CHUNK = 128
NCHUNKS = ROWS_PER_W // IDX_CHUNK


def _e2_body(cb_ref, e2_ref):
    cb = cb_ref[0]
    ones = jnp.ones((1, DHEAD), jnp.float32)
    e2_ref[0] = lax.dot_general(ones, cb * cb, (((1,), (1,)), ((), ())),
                                preferred_element_type=jnp.float32,
                                precision=lax.Precision.HIGHEST)


def _compute_e2(codebook):
    return pl.pallas_call(
        _e2_body,
        grid=(HEADS,),
        in_specs=[pl.BlockSpec((1, K, DHEAD), lambda h: (h, 0, 0))],
        out_specs=pl.BlockSpec((1, 1, K), lambda h: (h, 0, 0)),
        out_shape=jax.ShapeDtypeStruct((HEADS, 1, K), jnp.float32),
    )(codebook)


def _argmin_body(x_ref, cb_ref, e2_ref, idx_ref, val_scr, idx_scr):
    k = pl.program_id(1)

    @pl.when(k == 0)
    def _init():
        val_scr[...] = jnp.full((HEADS, N, LANES), jnp.inf, jnp.float32)
        idx_scr[...] = jnp.zeros((HEADS, N, LANES), jnp.int32)

    lane_iota = lax.broadcasted_iota(jnp.int32, (N, LANES), 1)
    finals = []
    for h in range(HEADS):
        xh = x_ref[0, :, h * DHEAD:(h + 1) * DHEAD]
        cbh = cb_ref[h]
        xen = lax.dot_general(xh, cbh, (((1,), (1,)), ((), ())),
                              preferred_element_type=jnp.float32)
        run_val = val_scr[h]
        run_idx = idx_scr[h]
        for j in range(SUBTILES):
            sj = (xen[:, j * LANES:(j + 1) * LANES]
                  + e2_ref[h, :, j * LANES:(j + 1) * LANES])
            cand = lane_iota + (k * KBLK + j * LANES)
            better = sj < run_val
            run_val = jnp.where(better, sj, run_val)
            run_idx = jnp.where(better, cand, run_idx)
        val_scr[h] = run_val
        idx_scr[h] = run_idx

        @pl.when(k == KSTEPS - 1)
        def _final(h=h, run_val=run_val, run_idx=run_idx):
            rowmin = jnp.min(run_val, axis=1, keepdims=True)
            big = jnp.int32(1 << 30)
            best = jnp.min(jnp.where(run_val == rowmin, run_idx, big),
                           axis=1, keepdims=True)
            idx_ref[0, h] = best + h * K


def _compute_indices(xneg2, cb_bf, e2):
    return pl.pallas_call(
        _argmin_body,
        grid=(B, KSTEPS),
        in_specs=[
            pl.BlockSpec((1, N, DIM), lambda b, k: (b, 0, 0)),
            pl.BlockSpec((HEADS, KBLK, DHEAD), lambda b, k: (0, k, 0)),
            pl.BlockSpec((HEADS, 1, KBLK), lambda b, k: (0, 0, k)),
        ],
        out_specs=pl.BlockSpec((1, HEADS, N, 1), lambda b, k: (b, 0, 0, 0)),
        out_shape=jax.ShapeDtypeStruct((B, HEADS, N, 1), jnp.int32),
        scratch_shapes=[
            pltpu.VMEM((HEADS, N, LANES), jnp.float32),
            pltpu.VMEM((HEADS, N, LANES), jnp.int32),
        ],
    )(xneg2, cb_bf, e2)


def _gather_body(table_hbm, idx_hbm, out_hbm, idx_v, rows_v, sem):
    wid = lax.axis_index("s") * 2 + lax.axis_index("c")
    pltpu.sync_copy(idx_hbm.at[wid], idx_v)
    copies = []
    for j in range(NCHUNKS):
        copies.append(pltpu.async_copy(
            table_hbm.at[idx_v.at[j]],
            rows_v.at[pl.ds(j * IDX_CHUNK, IDX_CHUNK)],
            sem))
    for c in copies:
        c.wait()
    pltpu.sync_copy(rows_v, out_hbm.at[pl.ds(wid * ROWS_PER_W, ROWS_PER_W)])


@functools.cache
def _gather_rows_kernel():
    return pl.kernel(
        _gather_body,
        out_type=jax.ShapeDtypeStruct((ROWS_TOTAL, DMEM), jnp.float32),
        mesh=plsc.VectorSubcoreMesh(core_axis_name="c", subcore_axis_name="s",
                                    num_cores=2, num_subcores=16),
        scratch_types=[
            pltpu.VMEM((NCHUNKS, IDX_CHUNK), jnp.int32),
            pltpu.VMEM((ROWS_PER_W, DMEM), jnp.float32),
            pltpu.SemaphoreType.DMA,
        ],
        compiler_params=pltpu.CompilerParams(use_tc_tiling_on_sc=False),
    )


def kernel(x, codebook, values):
    xneg2 = (-2.0 * x).astype(jnp.bfloat16)
    cb_bf = codebook.astype(jnp.bfloat16)
    e2 = _compute_e2(codebook)
    idx = _compute_indices(xneg2, cb_bf, e2)
    idx_flat = idx.reshape(B, HEADS, N).transpose(0, 2, 1)
    idx3 = idx_flat.reshape(NWORKERS, NCHUNKS, IDX_CHUNK)
    table = values.reshape(HEADS * K, DMEM)
    rows = _gather_rows_kernel()(table, idx3)
    return rows.reshape(B, N, HEADS * DMEM)


# speedup vs baseline: 2.5688x; 1.1901x over previous
import functools

import jax
import jax.numpy as jnp
from jax import lax
from jax.experimental import pallas as pl
from jax.experimental.pallas import tpu as pltpu
import jax.experimental.pallas.tpu_sc as plsc

DIM = 768
HEADS = 8
K = 8192
DHEAD = 96
DMEM = 96
B = 8
N = 576

HPAD = 128
KBLK = 2048
KSTEPS = K // KBLK
LANES = 128
SUBTILES = KBLK // LANES

NWORKERS = 32
ROWS_TOTAL = B * N * HEADS
ROWS_PER_W = ROWS_TOTAL // NWORKERS
IDX_CHUNK = 128
NCHUNKS = ROWS_PER_W // IDX_CHUNK


def _prep_body(cb_ref, x_ref, cbaug_ref, xaug_ref):
    cb = cb_ref[0]
    e2 = jnp.sum(cb * cb, axis=1, keepdims=True)
    hi = e2.astype(jnp.bfloat16)
    r1 = e2 - hi.astype(jnp.float32)
    mid = r1.astype(jnp.bfloat16)
    lo = (r1 - mid.astype(jnp.float32)).astype(jnp.bfloat16)
    zpad = jnp.zeros((K, HPAD - DHEAD - 3), jnp.bfloat16)
    cbaug_ref[0] = jnp.concatenate(
        [cb.astype(jnp.bfloat16), hi, mid, lo, zpad], axis=1)

    xb = (-2.0 * x_ref[0]).astype(jnp.bfloat16)
    ones3 = jnp.ones((N, 3), jnp.bfloat16)
    zx = jnp.zeros((N, HPAD - DHEAD - 3), jnp.bfloat16)
    parts = []
    for h in range(HEADS):
        parts += [xb[:, h * DHEAD:(h + 1) * DHEAD], ones3, zx]
    xaug_ref[0] = jnp.concatenate(parts, axis=1)


def _prep(codebook, x):
    return pl.pallas_call(
        _prep_body,
        grid=(HEADS,),
        in_specs=[
            pl.BlockSpec((1, K, DHEAD), lambda g: (g, 0, 0)),
            pl.BlockSpec((1, N, DIM), lambda g: (g, 0, 0)),
        ],
        out_specs=[
            pl.BlockSpec((1, K, HPAD), lambda g: (g, 0, 0)),
            pl.BlockSpec((1, N, HEADS * HPAD), lambda g: (g, 0, 0)),
        ],
        out_shape=[
            jax.ShapeDtypeStruct((HEADS, K, HPAD), jnp.bfloat16),
            jax.ShapeDtypeStruct((B, N, HEADS * HPAD), jnp.bfloat16),
        ],
    )(codebook, x)


def _argmin_body(x_ref, cb_ref, idx_ref, val_scr, idx_scr):
    k = pl.program_id(1)

    @pl.when(k == 0)
    def _init():
        val_scr[...] = jnp.full((HEADS, N, LANES), jnp.inf, jnp.float32)
        idx_scr[...] = jnp.zeros((HEADS, N, LANES), jnp.int32)

    lane_iota = lax.broadcasted_iota(jnp.int32, (N, LANES), 1)
    for h in range(HEADS):
        xh = x_ref[0, :, h * HPAD:(h + 1) * HPAD]
        cbh = cb_ref[h]
        scores = lax.dot_general(xh, cbh, (((1,), (1,)), ((), ())),
                                 preferred_element_type=jnp.float32)
        run_val = val_scr[h]
        run_idx = idx_scr[h]
        for j in range(SUBTILES):
            sj = scores[:, j * LANES:(j + 1) * LANES]
            cand = lane_iota + (k * KBLK + j * LANES)
            better = sj < run_val
            run_val = jnp.where(better, sj, run_val)
            run_idx = jnp.where(better, cand, run_idx)
        val_scr[h] = run_val
        idx_scr[h] = run_idx

        @pl.when(k == KSTEPS - 1)
        def _final(h=h, run_val=run_val, run_idx=run_idx):
            rowmin = jnp.min(run_val, axis=1, keepdims=True)
            big = jnp.int32(1 << 30)
            best = jnp.min(jnp.where(run_val == rowmin, run_idx, big),
                           axis=1, keepdims=True)
            idx_ref[0, h] = best + h * K


def _compute_indices(xaug, cbaug):
    return pl.pallas_call(
        _argmin_body,
        grid=(B, KSTEPS),
        in_specs=[
            pl.BlockSpec((1, N, HEADS * HPAD), lambda b, k: (b, 0, 0)),
            pl.BlockSpec((HEADS, KBLK, HPAD), lambda b, k: (0, k, 0)),
        ],
        out_specs=pl.BlockSpec((1, HEADS, N, 1), lambda b, k: (b, 0, 0, 0)),
        out_shape=jax.ShapeDtypeStruct((B, HEADS, N, 1), jnp.int32),
        scratch_shapes=[
            pltpu.VMEM((HEADS, N, LANES), jnp.float32),
            pltpu.VMEM((HEADS, N, LANES), jnp.int32),
        ],
    )(xaug, cbaug)


def _gather_body(table_hbm, idx_hbm, out_hbm, idx_v, rows_v, sem):
    wid = lax.axis_index("s") * 2 + lax.axis_index("c")
    pltpu.sync_copy(idx_hbm.at[wid], idx_v)
    copies = []
    for j in range(NCHUNKS):
        copies.append(pltpu.async_copy(
            table_hbm.at[idx_v.at[j]],
            rows_v.at[pl.ds(j * IDX_CHUNK, IDX_CHUNK)],
            sem))
    for c in copies:
        c.wait()
    pltpu.sync_copy(rows_v, out_hbm.at[pl.ds(wid * ROWS_PER_W, ROWS_PER_W)])


@functools.cache
def _gather_rows_kernel():
    return pl.kernel(
        _gather_body,
        out_type=jax.ShapeDtypeStruct((ROWS_TOTAL, DMEM), jnp.float32),
        mesh=plsc.VectorSubcoreMesh(core_axis_name="c", subcore_axis_name="s",
                                    num_cores=2, num_subcores=16),
        scratch_types=[
            pltpu.VMEM((NCHUNKS, IDX_CHUNK), jnp.int32),
            pltpu.VMEM((ROWS_PER_W, DMEM), jnp.float32),
            pltpu.SemaphoreType.DMA,
        ],
        compiler_params=pltpu.CompilerParams(use_tc_tiling_on_sc=False),
    )


def kernel(x, codebook, values):
    cbaug, xaug = _prep(codebook, x)
    idx = _compute_indices(xaug, cbaug)
    idx_flat = idx.reshape(B, HEADS, N).transpose(0, 2, 1)
    idx3 = idx_flat.reshape(NWORKERS, NCHUNKS, IDX_CHUNK)
    table = values.reshape(HEADS * K, DMEM)
    rows = _gather_rows_kernel()(table, idx3)
    return rows.reshape(B, N, HEADS * DMEM)


# speedup vs baseline: 2.9582x; 1.1516x over previous
import functools

import jax
import jax.numpy as jnp
from jax import lax
from jax.experimental import pallas as pl
from jax.experimental.pallas import tpu as pltpu
import jax.experimental.pallas.tpu_sc as plsc

DIM = 768
HEADS = 8
K = 8192
DHEAD = 96
DMEM = 96
B = 8
N = 576

HPAD = 128
KBLK = 2048
KSTEPS = K // KBLK
LANES = 128
SUBTILES = KBLK // LANES

NWORKERS = 32
ROWS_TOTAL = B * N * HEADS
ROWS_PER_W = ROWS_TOTAL // NWORKERS
IDX_CHUNK = 128
NCHUNKS = ROWS_PER_W // IDX_CHUNK


def _prep_body(cb_ref, x_ref, cbaug_ref, xaug_ref):
    cb = cb_ref[0]
    e2 = jnp.sum(cb * cb, axis=1, keepdims=True)
    hi = e2.astype(jnp.bfloat16)
    r1 = e2 - hi.astype(jnp.float32)
    mid = r1.astype(jnp.bfloat16)
    lo = (r1 - mid.astype(jnp.float32)).astype(jnp.bfloat16)
    zpad = jnp.zeros((K, HPAD - DHEAD - 3), jnp.bfloat16)
    cbaug_ref[0] = jnp.concatenate(
        [cb.astype(jnp.bfloat16), hi, mid, lo, zpad], axis=1)

    xb = (-2.0 * x_ref[0]).astype(jnp.bfloat16)
    ones3 = jnp.ones((N, 3), jnp.bfloat16)
    zx = jnp.zeros((N, HPAD - DHEAD - 3), jnp.bfloat16)
    parts = []
    for h in range(HEADS):
        parts += [xb[:, h * DHEAD:(h + 1) * DHEAD], ones3, zx]
    xaug_ref[0] = jnp.concatenate(parts, axis=1)


def _prep(codebook, x):
    return pl.pallas_call(
        _prep_body,
        grid=(HEADS,),
        in_specs=[
            pl.BlockSpec((1, K, DHEAD), lambda g: (g, 0, 0)),
            pl.BlockSpec((1, N, DIM), lambda g: (g, 0, 0)),
        ],
        out_specs=[
            pl.BlockSpec((1, K, HPAD), lambda g: (g, 0, 0)),
            pl.BlockSpec((1, N, HEADS * HPAD), lambda g: (g, 0, 0)),
        ],
        out_shape=[
            jax.ShapeDtypeStruct((HEADS, K, HPAD), jnp.bfloat16),
            jax.ShapeDtypeStruct((B, N, HEADS * HPAD), jnp.bfloat16),
        ],
    )(codebook, x)


def _argmin_body(x_ref, cb_ref, idx_ref, val_scr, idx_scr):
    k = pl.program_id(1)

    @pl.when(k == 0)
    def _init():
        val_scr[...] = jnp.full((HEADS, N, LANES), jnp.inf, jnp.float32)
        idx_scr[...] = jnp.zeros((HEADS, N, LANES), jnp.int32)

    lane_iota = lax.broadcasted_iota(jnp.int32, (N, LANES), 1)
    for h in range(HEADS):
        xh = x_ref[0, :, h * HPAD:(h + 1) * HPAD]
        cbh = cb_ref[h]
        scores = lax.dot_general(xh, cbh, (((1,), (1,)), ((), ())),
                                 preferred_element_type=jnp.float32)
        run_val = val_scr[h]
        run_idx = idx_scr[h]
        for j in range(SUBTILES):
            sj = scores[:, j * LANES:(j + 1) * LANES]
            cand = lane_iota + (k * KBLK + j * LANES)
            better = sj < run_val
            run_val = jnp.where(better, sj, run_val)
            run_idx = jnp.where(better, cand, run_idx)
        val_scr[h] = run_val
        idx_scr[h] = run_idx

    @pl.when(k == KSTEPS - 1)
    def _final():
        big = jnp.int32(1 << 30)
        for h in range(HEADS):
            run_val = val_scr[h]
            run_idx = idx_scr[h]
            rowmin = jnp.min(run_val, axis=1, keepdims=True)
            best = jnp.min(jnp.where(run_val == rowmin, run_idx, big),
                           axis=1, keepdims=True)
            idx_ref[0, h] = best + h * K


def _compute_indices(xaug, cbaug):
    return pl.pallas_call(
        _argmin_body,
        grid=(B, KSTEPS),
        in_specs=[
            pl.BlockSpec((1, N, HEADS * HPAD), lambda b, k: (b, 0, 0)),
            pl.BlockSpec((HEADS, KBLK, HPAD), lambda b, k: (0, k, 0)),
        ],
        out_specs=pl.BlockSpec((1, HEADS, N, 1), lambda b, k: (b, 0, 0, 0)),
        out_shape=jax.ShapeDtypeStruct((B, HEADS, N, 1), jnp.int32),
        scratch_shapes=[
            pltpu.VMEM((HEADS, N, LANES), jnp.float32),
            pltpu.VMEM((HEADS, N, LANES), jnp.int32),
        ],
    )(xaug, cbaug)


def _gather_body(table_hbm, idx_hbm, out_hbm, idx_v, rows_v, sem):
    wid = lax.axis_index("s") * 2 + lax.axis_index("c")
    pltpu.sync_copy(idx_hbm.at[wid], idx_v)
    copies = []
    for j in range(NCHUNKS):
        copies.append(pltpu.async_copy(
            table_hbm.at[idx_v.at[j]],
            rows_v.at[pl.ds(j * IDX_CHUNK, IDX_CHUNK)],
            sem))
    for c in copies:
        c.wait()
    pltpu.sync_copy(rows_v, out_hbm.at[pl.ds(wid * ROWS_PER_W, ROWS_PER_W)])


@functools.cache
def _gather_rows_kernel():
    return pl.kernel(
        _gather_body,
        out_type=jax.ShapeDtypeStruct((ROWS_TOTAL, DMEM), jnp.float32),
        mesh=plsc.VectorSubcoreMesh(core_axis_name="c", subcore_axis_name="s",
                                    num_cores=2, num_subcores=16),
        scratch_types=[
            pltpu.VMEM((NCHUNKS, IDX_CHUNK), jnp.int32),
            pltpu.VMEM((ROWS_PER_W, DMEM), jnp.float32),
            pltpu.SemaphoreType.DMA,
        ],
        compiler_params=pltpu.CompilerParams(use_tc_tiling_on_sc=False),
    )


def kernel(x, codebook, values):
    cbaug, xaug = _prep(codebook, x)
    idx = _compute_indices(xaug, cbaug)
    idx_flat = idx.reshape(B, HEADS, N).transpose(0, 2, 1)
    idx3 = idx_flat.reshape(NWORKERS, NCHUNKS, IDX_CHUNK)
    table = values.reshape(HEADS * K, DMEM)
    rows = _gather_rows_kernel()(table, idx3)
    return rows.reshape(B, N, HEADS * DMEM)


# speedup vs baseline: 3.0503x; 1.0311x over previous
import functools

import jax
import jax.numpy as jnp
from jax import lax
from jax.experimental import pallas as pl
from jax.experimental.pallas import tpu as pltpu
import jax.experimental.pallas.tpu_sc as plsc

DIM = 768
HEADS = 8
K = 8192
DHEAD = 96
DMEM = 96
B = 8
N = 576

HPAD = 128
KBLK = 4096
KSTEPS = K // KBLK
LANES = 128
SUBTILES = KBLK // LANES

NWORKERS = 32
ROWS_TOTAL = B * N * HEADS
ROWS_PER_W = ROWS_TOTAL // NWORKERS
IDX_CHUNK = 128
NCHUNKS = ROWS_PER_W // IDX_CHUNK


def _prep_body(cb_ref, x_ref, cbaug_ref, xaug_ref):
    cb = cb_ref[0]
    e2 = jnp.sum(cb * cb, axis=1, keepdims=True)
    hi = e2.astype(jnp.bfloat16)
    r1 = e2 - hi.astype(jnp.float32)
    mid = r1.astype(jnp.bfloat16)
    lo = (r1 - mid.astype(jnp.float32)).astype(jnp.bfloat16)
    zpad = jnp.zeros((K, HPAD - DHEAD - 3), jnp.bfloat16)
    cbaug_ref[0] = jnp.concatenate(
        [cb.astype(jnp.bfloat16), hi, mid, lo, zpad], axis=1)

    xb = (-2.0 * x_ref[0]).astype(jnp.bfloat16)
    ones3 = jnp.ones((N, 3), jnp.bfloat16)
    zx = jnp.zeros((N, HPAD - DHEAD - 3), jnp.bfloat16)
    parts = []
    for h in range(HEADS):
        parts += [xb[:, h * DHEAD:(h + 1) * DHEAD], ones3, zx]
    xaug_ref[0] = jnp.concatenate(parts, axis=1)


def _prep(codebook, x):
    return pl.pallas_call(
        _prep_body,
        grid=(HEADS,),
        in_specs=[
            pl.BlockSpec((1, K, DHEAD), lambda g: (g, 0, 0)),
            pl.BlockSpec((1, N, DIM), lambda g: (g, 0, 0)),
        ],
        out_specs=[
            pl.BlockSpec((1, K, HPAD), lambda g: (g, 0, 0)),
            pl.BlockSpec((1, N, HEADS * HPAD), lambda g: (g, 0, 0)),
        ],
        out_shape=[
            jax.ShapeDtypeStruct((HEADS, K, HPAD), jnp.bfloat16),
            jax.ShapeDtypeStruct((B, N, HEADS * HPAD), jnp.bfloat16),
        ],
    )(codebook, x)


def _argmin_body(x_ref, cb_ref, idx_ref, val_scr, idx_scr):
    k = pl.program_id(1)

    @pl.when(k == 0)
    def _init():
        val_scr[...] = jnp.full((HEADS, N, LANES), jnp.inf, jnp.float32)
        idx_scr[...] = jnp.zeros((HEADS, N, LANES), jnp.int32)

    lane_iota = lax.broadcasted_iota(jnp.int32, (N, LANES), 1)
    for h in range(HEADS):
        xh = x_ref[0, :, h * HPAD:(h + 1) * HPAD]
        cbh = cb_ref[h]
        scores = lax.dot_general(xh, cbh, (((1,), (1,)), ((), ())),
                                 preferred_element_type=jnp.float32)
        run_val = val_scr[h]
        run_idx = idx_scr[h]
        for j in range(SUBTILES):
            sj = scores[:, j * LANES:(j + 1) * LANES]
            cand = lane_iota + (k * KBLK + j * LANES)
            better = sj < run_val
            run_val = jnp.minimum(sj, run_val)
            run_idx = jnp.where(better, cand, run_idx)
        val_scr[h] = run_val
        idx_scr[h] = run_idx

    @pl.when(k == KSTEPS - 1)
    def _final():
        big = jnp.int32(1 << 30)
        cols = []
        for h in range(HEADS):
            run_val = val_scr[h]
            run_idx = idx_scr[h]
            rowmin = jnp.min(run_val, axis=1, keepdims=True)
            best = jnp.min(jnp.where(run_val == rowmin, run_idx, big),
                           axis=1, keepdims=True)
            cols.append(best + h * K)
        idx_ref[0] = jnp.concatenate(cols, axis=1)


def _compute_indices(xaug, cbaug):
    return pl.pallas_call(
        _argmin_body,
        grid=(B, KSTEPS),
        in_specs=[
            pl.BlockSpec((1, N, HEADS * HPAD), lambda b, k: (b, 0, 0)),
            pl.BlockSpec((HEADS, KBLK, HPAD), lambda b, k: (0, k, 0)),
        ],
        out_specs=pl.BlockSpec((1, N, HEADS), lambda b, k: (b, 0, 0)),
        out_shape=jax.ShapeDtypeStruct((B, N, HEADS), jnp.int32),
        scratch_shapes=[
            pltpu.VMEM((HEADS, N, LANES), jnp.float32),
            pltpu.VMEM((HEADS, N, LANES), jnp.int32),
        ],
        compiler_params=pltpu.CompilerParams(vmem_limit_bytes=100 * 1024 * 1024),
    )(xaug, cbaug)


def _gather_body(table_hbm, idx_hbm, out_hbm, idx_v, rows_v, sem):
    wid = lax.axis_index("s") * 2 + lax.axis_index("c")
    pltpu.sync_copy(idx_hbm.at[wid], idx_v)
    copies = []
    for j in range(NCHUNKS):
        copies.append(pltpu.async_copy(
            table_hbm.at[idx_v.at[j]],
            rows_v.at[pl.ds(j * IDX_CHUNK, IDX_CHUNK)],
            sem))
    for c in copies:
        c.wait()
    pltpu.sync_copy(rows_v, out_hbm.at[pl.ds(wid * ROWS_PER_W, ROWS_PER_W)])


@functools.cache
def _gather_rows_kernel():
    return pl.kernel(
        _gather_body,
        out_type=jax.ShapeDtypeStruct((ROWS_TOTAL, DMEM), jnp.float32),
        mesh=plsc.VectorSubcoreMesh(core_axis_name="c", subcore_axis_name="s",
                                    num_cores=2, num_subcores=16),
        scratch_types=[
            pltpu.VMEM((NCHUNKS, IDX_CHUNK), jnp.int32),
            pltpu.VMEM((ROWS_PER_W, DMEM), jnp.float32),
            pltpu.SemaphoreType.DMA,
        ],
        compiler_params=pltpu.CompilerParams(use_tc_tiling_on_sc=False),
    )


def kernel(x, codebook, values):
    cbaug, xaug = _prep(codebook, x)
    idx = _compute_indices(xaug, cbaug)
    idx3 = idx.reshape(NWORKERS, NCHUNKS, IDX_CHUNK)
    table = values.reshape(HEADS * K, DMEM)
    rows = _gather_rows_kernel()(table, idx3)
    return rows.reshape(B, N, HEADS * DMEM)
